# baseline (speedup 1.0000x reference)
import functools

import jax
import jax.numpy as jnp
import numpy as np
from jax import lax
from jax.experimental import pallas as pl
from jax.experimental.pallas import tpu as pltpu
from jax.experimental.pallas import tpu_sc as plsc

TABLE_SIZE = 1000000
MEM_DIM = 128
DIM = 2048

_NC = 2
_NS = 16
_NW = _NC * _NS

_BLOCK_ROWS = 1024
_SC_SUB = 2

_PACK_PERM = np.empty(MEM_DIM, np.int32)
for _g in range(MEM_DIM // 32):
    for _k in range(16):
        _PACK_PERM[32 * _g + 2 * _k] = 32 * _g + _k
        _PACK_PERM[32 * _g + 2 * _k + 1] = 32 * _g + 16 + _k


def _sc_gather_bf16(table, idx, n_rows):
    b_per_w = n_rows // _NW
    sub = b_per_w // _SC_SUB
    mesh = plsc.VectorSubcoreMesh(core_axis_name="c", subcore_axis_name="s")

    @functools.partial(
        pl.kernel,
        mesh=mesh,
        out_type=jax.ShapeDtypeStruct((n_rows, MEM_DIM), jnp.bfloat16),
        scratch_types=[
            pltpu.VMEM((b_per_w,), jnp.int32),
            pltpu.VMEM((b_per_w, MEM_DIM), jnp.float32),
            pltpu.VMEM((b_per_w, MEM_DIM), jnp.bfloat16),
            [pltpu.SemaphoreType.DMA] * _SC_SUB,
            pltpu.SemaphoreType.DMA,
        ],
        compiler_params=pltpu.CompilerParams(needs_layout_passes=False),
    )
    def gather_kernel(table_hbm, idx_hbm, out_hbm, idx_v, rows_v, bf_v,
                      gsems, wsem):
        wid = lax.axis_index("s") * _NC + lax.axis_index("c")
        base = wid * b_per_w
        pltpu.sync_copy(idx_hbm.at[pl.ds(base, b_per_w)], idx_v)
        gathers = [
            pltpu.async_copy(
                table_hbm.at[idx_v.at[pl.ds(s * sub, sub)]],
                rows_v.at[pl.ds(s * sub, sub)], gsems[s])
            for s in range(_SC_SUB)
        ]
        writes = []
        for s in range(_SC_SUB):
            gathers[s].wait()

            def pack_row(r, carry, s=s):
                row = s * sub + r
                for c in range(MEM_DIM // 32):
                    a = rows_v[row, pl.ds(c * 32, 16)]
                    b = rows_v[row, pl.ds(c * 32 + 16, 16)]
                    bf_v[row, pl.ds(c * 32, 32)] = plsc.pack(
                        a, b, format=plsc.PackFormat.INTERLEAVED)
                return carry

            lax.fori_loop(0, sub, pack_row, 0)
            writes.append(pltpu.async_copy(
                bf_v.at[pl.ds(s * sub, sub)],
                out_hbm.at[pl.ds(base + s * sub, sub)], wsem))
        for w in writes:
            w.wait()

    return gather_kernel(table, idx)


def _mm_body(ids_ref, mem_ref, w_ref, out_ref):
    mask = (ids_ref[0, 0, :] != 0).astype(jnp.bfloat16)
    mem = mem_ref[...] * mask[:, None]
    out_ref[...] = lax.dot_general(
        mem, w_ref[...], (((1,), (1,)), ((), ())),
        preferred_element_type=jnp.float32)


def _tc_matmul(mem, w, ids3, n_rows):
    grid = (n_rows // _BLOCK_ROWS,)
    return pl.pallas_call(
        _mm_body,
        grid=grid,
        in_specs=[
            pl.BlockSpec((1, 1, _BLOCK_ROWS), lambda i: (i, 0, 0)),
            pl.BlockSpec((_BLOCK_ROWS, MEM_DIM), lambda i: (i, 0)),
            pl.BlockSpec((DIM, MEM_DIM), lambda i: (0, 0)),
        ],
        out_specs=pl.BlockSpec((_BLOCK_ROWS, DIM), lambda i: (i, 0)),
        out_shape=jax.ShapeDtypeStruct((n_rows, DIM), jnp.float32),
    )(ids3, mem, w)


def kernel(x, bigram_ids, layer_id, collect_stats, E, W):
    b, s = bigram_ids.shape
    n_rows = b * s
    ids = bigram_ids.reshape(n_rows).astype(jnp.int32)
    mem_bf = _sc_gather_bf16(E, ids, n_rows)
    w_bf = W[:, _PACK_PERM].astype(jnp.bfloat16)
    ids3 = ids.reshape(n_rows // _BLOCK_ROWS, 1, _BLOCK_ROWS)
    out = _tc_matmul(mem_bf, w_bf, ids3, n_rows)
    return out.reshape(b, s, DIM)


# speedup vs baseline: 1.0929x; 1.0929x over previous
import functools

import jax
import jax.numpy as jnp
from jax import lax
from jax.experimental import pallas as pl
from jax.experimental.pallas import tpu as pltpu
from jax.experimental.pallas import tpu_sc as plsc

TABLE_SIZE = 1000000
MEM_DIM = 128
DIM = 2048

_NC = 2
_NS = 16
_NW = _NC * _NS

_BLOCK_ROWS = 1024
_SC_SUB = 2


def _sc_gather(table, idx, n_rows):
    b_per_w = n_rows // _NW
    sub = b_per_w // _SC_SUB
    mesh = plsc.VectorSubcoreMesh(core_axis_name="c", subcore_axis_name="s")

    @functools.partial(
        pl.kernel,
        mesh=mesh,
        out_type=jax.ShapeDtypeStruct((n_rows, MEM_DIM), jnp.float32),
        scratch_types=[
            pltpu.VMEM((b_per_w,), jnp.int32),
            pltpu.VMEM((b_per_w, MEM_DIM), jnp.float32),
            [pltpu.SemaphoreType.DMA] * _SC_SUB,
            pltpu.SemaphoreType.DMA,
        ],
    )
    def gather_kernel(table_hbm, idx_hbm, out_hbm, idx_v, rows_v, gsems, wsem):
        wid = lax.axis_index("s") * _NC + lax.axis_index("c")
        base = wid * b_per_w
        pltpu.sync_copy(idx_hbm.at[pl.ds(base, b_per_w)], idx_v)
        gathers = [
            pltpu.async_copy(
                table_hbm.at[idx_v.at[pl.ds(s * sub, sub)]],
                rows_v.at[pl.ds(s * sub, sub)], gsems[s])
            for s in range(_SC_SUB)
        ]
        writes = []
        for s in range(_SC_SUB):
            gathers[s].wait()
            writes.append(pltpu.async_copy(
                rows_v.at[pl.ds(s * sub, sub)],
                out_hbm.at[pl.ds(base + s * sub, sub)], wsem))
        for w in writes:
            w.wait()

    return gather_kernel(table, idx)


def _mm_body(ids_ref, mem_ref, w_ref, out_ref):
    mask = (ids_ref[0, 0, :] != 0).astype(jnp.float32)
    mem = mem_ref[...] * mask[:, None]
    out_ref[...] = lax.dot_general(
        mem, w_ref[...], (((1,), (1,)), ((), ())),
        preferred_element_type=jnp.float32)


def _tc_matmul(mem, w, ids3, n_rows):
    grid = (n_rows // _BLOCK_ROWS,)
    return pl.pallas_call(
        _mm_body,
        grid=grid,
        in_specs=[
            pl.BlockSpec((1, 1, _BLOCK_ROWS), lambda i: (i, 0, 0)),
            pl.BlockSpec((_BLOCK_ROWS, MEM_DIM), lambda i: (i, 0)),
            pl.BlockSpec((DIM, MEM_DIM), lambda i: (0, 0)),
        ],
        out_specs=pl.BlockSpec((_BLOCK_ROWS, DIM), lambda i: (i, 0)),
        out_shape=jax.ShapeDtypeStruct((n_rows, DIM), jnp.float32),
    )(ids3, mem, w)


def kernel(x, bigram_ids, layer_id, collect_stats, E, W):
    b, s = bigram_ids.shape
    n_rows = b * s
    ids = bigram_ids.reshape(n_rows).astype(jnp.int32)
    mem = _sc_gather(E, ids, n_rows)
    ids3 = ids.reshape(n_rows // _BLOCK_ROWS, 1, _BLOCK_ROWS)
    out = _tc_matmul(mem, W, ids3, n_rows)
    return out.reshape(b, s, DIM)
